# baseline (speedup 1.0000x reference)
import functools

import jax
import jax.numpy as jnp
from jax import lax
from jax.experimental import pallas as pl
from jax.experimental.pallas import tpu as pltpu
from jax.experimental.pallas import tpu_sc as plsc

_NC = 2
_NS = 16
_NW = _NC * _NS
_L = 16


def _make_kernel(n):
    chunk = (n // (_NW * _L)) * _L
    tail_vecs = (n - _NW * chunk) // _L
    assert chunk > 0 and _NW * chunk + tail_vecs * _L == n
    tail_base = _NW * chunk
    half1 = (chunk // (2 * _L)) * _L
    half2 = chunk - half1

    mesh = plsc.VectorSubcoreMesh(core_axis_name="c", subcore_axis_name="s")

    @functools.partial(
        pl.kernel,
        out_type=jax.ShapeDtypeStruct((n,), jnp.float32),
        mesh=mesh,
        compiler_params=pltpu.CompilerParams(needs_layout_passes=False),
        scratch_types=[
            pltpu.VMEM((chunk,), jnp.float32),
            pltpu.VMEM((chunk,), jnp.int32),
            pltpu.VMEM((chunk,), jnp.float32),
            pltpu.VMEM((_L,), jnp.float32),
            pltpu.VMEM((_L,), jnp.float32),
            pltpu.VMEM((_L,), jnp.float32),
            pltpu.VMEM((_L,), jnp.int32),
            pltpu.VMEM((_L,), jnp.float32),
            pltpu.SemaphoreType.DMA,
            pltpu.SemaphoreType.DMA,
            pltpu.SemaphoreType.DMA,
        ],
    )
    def rescale(x_hbm, t_hbm, scale_hbm, shift_hbm, out_hbm,
                x_v, t_v, o_v, scale_v, shift_v, xt_v, tt_v, ot_v,
                sem_a, sem_b, sem_c):
        wid = lax.axis_index("s") * _NC + lax.axis_index("c")
        base = wid * chunk
        mid = base + half1

        first = [
            pltpu.async_copy(scale_hbm, scale_v, sem_a),
            pltpu.async_copy(shift_hbm, shift_v, sem_a),
            pltpu.async_copy(x_hbm.at[pl.ds(base, half1)],
                             x_v.at[pl.ds(0, half1)], sem_a),
            pltpu.async_copy(t_hbm.at[pl.ds(base, half1)],
                             t_v.at[pl.ds(0, half1)], sem_a),
        ]
        second = [
            pltpu.async_copy(x_hbm.at[pl.ds(mid, half2)],
                             x_v.at[pl.ds(half1, half2)], sem_b),
            pltpu.async_copy(t_hbm.at[pl.ds(mid, half2)],
                             t_v.at[pl.ds(half1, half2)], sem_b),
        ]
        if tail_vecs:
            @pl.when(wid < tail_vecs)
            def _():
                tb = tail_base + wid * _L
                cx = pltpu.async_copy(x_hbm.at[pl.ds(tb, _L)], xt_v, sem_b)
                ct = pltpu.async_copy(t_hbm.at[pl.ds(tb, _L)], tt_v, sem_b)
                cx.wait()
                ct.wait()

        def compute(off):
            idx = t_v[pl.ds(off, _L)]
            xv = x_v[pl.ds(off, _L)]
            s = plsc.load_gather(scale_v, [idx])
            b = plsc.load_gather(shift_v, [idx])
            o_v[pl.ds(off, _L)] = xv * s + b

        for c in first:
            c.wait()
        plsc.parallel_loop(0, half1, step=_L, unroll=8)(compute)
        store1 = pltpu.async_copy(o_v.at[pl.ds(0, half1)],
                                  out_hbm.at[pl.ds(base, half1)], sem_c)

        for c in second:
            c.wait()
        plsc.parallel_loop(half1, chunk, step=_L, unroll=8)(compute)
        store2 = pltpu.async_copy(o_v.at[pl.ds(half1, half2)],
                                  out_hbm.at[pl.ds(mid, half2)], sem_c)

        if tail_vecs:
            @pl.when(wid < tail_vecs)
            def _():
                tb = tail_base + wid * _L
                idx = tt_v[...]
                s = plsc.load_gather(scale_v, [idx])
                b = plsc.load_gather(shift_v, [idx])
                ot_v[...] = xt_v[...] * s + b
                pltpu.async_copy(ot_v, out_hbm.at[pl.ds(tb, _L)], sem_c)

        store1.wait()
        store2.wait()
        if tail_vecs:
            @pl.when(wid < tail_vecs)
            def _():
                tb = tail_base + wid * _L
                pltpu.make_async_copy(
                    ot_v, out_hbm.at[pl.ds(tb, _L)], sem_c).wait()

    return rescale


def kernel(scaled_atomic_energy, atom_type, scale, shift):
    n = scaled_atomic_energy.shape[0]
    x = scaled_atomic_energy.reshape(n)
    t = atom_type.astype(jnp.int32)
    out = _make_kernel(n)(x, t, scale, shift)
    return out.reshape(n, 1)


# speedup vs baseline: 1.0111x; 1.0111x over previous
import functools

import jax
import jax.numpy as jnp
from jax import lax
from jax.experimental import pallas as pl
from jax.experimental.pallas import tpu as pltpu
from jax.experimental.pallas import tpu_sc as plsc

_NC = 2
_NS = 16
_NW = _NC * _NS
_L = 16


def _make_kernel(n):
    assert n % _L == 0
    nvec = n // _L
    vmain = -(-nvec // _NW)
    vlast = nvec - (_NW - 1) * vmain
    assert 0 < vlast <= vmain
    cmain = vmain * _L
    clast = vlast * _L

    mesh = plsc.VectorSubcoreMesh(core_axis_name="c", subcore_axis_name="s")

    @functools.partial(
        pl.kernel,
        out_type=jax.ShapeDtypeStruct((n,), jnp.float32),
        mesh=mesh,
        compiler_params=pltpu.CompilerParams(needs_layout_passes=False),
        scratch_types=[
            pltpu.VMEM((cmain,), jnp.float32),
            pltpu.VMEM((cmain,), jnp.int32),
            pltpu.VMEM((cmain,), jnp.float32),
            pltpu.VMEM((_L,), jnp.float32),
            pltpu.VMEM((_L,), jnp.float32),
            pltpu.SemaphoreType.DMA,
            pltpu.SemaphoreType.DMA,
        ],
    )
    def rescale(x_hbm, t_hbm, scale_hbm, shift_hbm, out_hbm,
                x_v, t_v, o_v, scale_v, shift_v, sem_in, sem_out):
        wid = lax.axis_index("s") * _NC + lax.axis_index("c")
        base = wid * cmain
        is_last = wid == _NW - 1

        cs = pltpu.async_copy(scale_hbm, scale_v, sem_in)
        cb = pltpu.async_copy(shift_hbm, shift_v, sem_in)
        main_in = [
            pltpu.make_async_copy(x_hbm.at[pl.ds(base, cmain)], x_v, sem_in),
            pltpu.make_async_copy(t_hbm.at[pl.ds(base, cmain)], t_v, sem_in),
        ]
        main_out = pltpu.make_async_copy(
            o_v, out_hbm.at[pl.ds(base, cmain)], sem_out)
        last_in = [
            pltpu.make_async_copy(x_hbm.at[pl.ds(base, clast)],
                                  x_v.at[pl.ds(0, clast)], sem_in),
            pltpu.make_async_copy(t_hbm.at[pl.ds(base, clast)],
                                  t_v.at[pl.ds(0, clast)], sem_in),
        ]
        last_out = pltpu.make_async_copy(
            o_v.at[pl.ds(0, clast)], out_hbm.at[pl.ds(base, clast)], sem_out)

        @pl.when(jnp.logical_not(is_last))
        def _():
            for c in main_in:
                c.start()

        @pl.when(is_last)
        def _():
            for c in last_in:
                c.start()

        cs.wait()
        cb.wait()

        @pl.when(jnp.logical_not(is_last))
        def _():
            for c in main_in:
                c.wait()

        @pl.when(is_last)
        def _():
            for c in last_in:
                c.wait()

        upper = jnp.where(is_last, clast, cmain)

        @plsc.parallel_loop(0, upper, step=_L, unroll=8)
        def _(off):
            idx = t_v[pl.ds(off, _L)]
            xv = x_v[pl.ds(off, _L)]
            s = plsc.load_gather(scale_v, [idx])
            b = plsc.load_gather(shift_v, [idx])
            o_v[pl.ds(off, _L)] = xv * s + b

        @pl.when(jnp.logical_not(is_last))
        def _():
            main_out.start()
            main_out.wait()

        @pl.when(is_last)
        def _():
            last_out.start()
            last_out.wait()

    return rescale


def kernel(scaled_atomic_energy, atom_type, scale, shift):
    n = scaled_atomic_energy.shape[0]
    x = scaled_atomic_energy.reshape(n)
    t = atom_type.astype(jnp.int32)
    out = _make_kernel(n)(x, t, scale, shift)
    return out.reshape(n, 1)
